# Initial kernel scaffold; baseline (speedup 1.0000x reference)
import jax
import jax.numpy as jnp
from jax.experimental import pallas as pl


def kernel(x, edge_feat, edge_index, W, b):
    raise NotImplementedError("write your pallas kernel here")


# speedup vs baseline: 2.1520x; 2.1520x over previous
import functools

import jax
import jax.numpy as jnp
from jax import lax
from jax.experimental import pallas as pl
from jax.experimental.pallas import tpu as pltpu
from jax.experimental.pallas import tpu_sc as plsc

N_NODES = 10000
N_EDGES = 160000
D_FEAT = 256
D_EDGE = 16
D_OUT = 256

NC, NS, LANES = 2, 16, 16
NW = NC * NS
EPW = N_EDGES // NW
CHUNK = 200
N_CHUNKS = EPW // CHUNK


def _gather_sum_body(xs_hbm, xt_hbm, isrc_hbm, itgt_hbm, out_hbm,
                     idx_v, rows_s, rows_t, sem):
    wid = lax.axis_index("s") * NC + lax.axis_index("c")

    def chunk_body(g, carry):
        base = wid * EPW + g * CHUNK
        pltpu.sync_copy(isrc_hbm.at[pl.ds(base, CHUNK)], idx_v)
        pltpu.async_copy(xs_hbm.at[idx_v], rows_s, sem).wait()
        pltpu.sync_copy(itgt_hbm.at[pl.ds(base, CHUNK)], idx_v)
        pltpu.async_copy(xt_hbm.at[idx_v], rows_t, sem).wait()

        def row_body(e, c2):
            for j in range(D_FEAT // LANES):
                sl = pl.ds(j * LANES, LANES)
                rows_s[e, sl] = rows_s[e, sl] + rows_t[e, sl]
            return c2

        lax.fori_loop(0, CHUNK, row_body, 0)
        pltpu.sync_copy(rows_s, out_hbm.at[pl.ds(base, CHUNK)])
        return carry

    lax.fori_loop(0, N_CHUNKS, chunk_body, 0)


_gather_sum = pl.kernel(
    _gather_sum_body,
    out_type=jax.ShapeDtypeStruct((N_EDGES, D_FEAT), jnp.float32),
    mesh=plsc.VectorSubcoreMesh(core_axis_name="c", subcore_axis_name="s"),
    scratch_types=[
        pltpu.VMEM((CHUNK,), jnp.int32),
        pltpu.VMEM((CHUNK, D_FEAT), jnp.float32),
        pltpu.VMEM((CHUNK, D_FEAT), jnp.float32),
        pltpu.SemaphoreType.DMA,
    ],
)


def _project_body(x_ref, ws_ref, wt_ref, xs_ref, xt_ref):
    xb = x_ref[...]
    xs_ref[...] = jnp.dot(xb, ws_ref[...], preferred_element_type=jnp.float32)
    xt_ref[...] = jnp.dot(xb, wt_ref[...], preferred_element_type=jnp.float32)


def _project(x, ws, wt):
    m_blk = 1000
    grid = (N_NODES // m_blk,)
    return pl.pallas_call(
        _project_body,
        grid=grid,
        in_specs=[
            pl.BlockSpec((m_blk, D_FEAT), lambda i: (i, 0)),
            pl.BlockSpec((D_FEAT, D_FEAT), lambda i: (0, 0)),
            pl.BlockSpec((D_FEAT, D_FEAT), lambda i: (0, 0)),
        ],
        out_specs=[
            pl.BlockSpec((m_blk, D_FEAT), lambda i: (i, 0)),
            pl.BlockSpec((m_blk, D_FEAT), lambda i: (i, 0)),
        ],
        out_shape=[
            jax.ShapeDtypeStruct((N_NODES, D_FEAT), jnp.float32),
            jax.ShapeDtypeStruct((N_NODES, D_FEAT), jnp.float32),
        ],
    )(x, ws, wt)


def _final_body(ef_ref, s_ref, we_ref, b_ref, o_ref):
    acc = jnp.dot(ef_ref[...], we_ref[...], preferred_element_type=jnp.float32)
    o_ref[...] = jnp.maximum(acc + s_ref[...] + b_ref[...], 0.0)


def _final(edge_feat, s, we, b2d):
    m_blk = 2000
    grid = (N_EDGES // m_blk,)
    return pl.pallas_call(
        _final_body,
        grid=grid,
        in_specs=[
            pl.BlockSpec((m_blk, D_EDGE), lambda i: (i, 0)),
            pl.BlockSpec((m_blk, D_OUT), lambda i: (i, 0)),
            pl.BlockSpec((D_EDGE, D_OUT), lambda i: (0, 0)),
            pl.BlockSpec((1, D_OUT), lambda i: (0, 0)),
        ],
        out_specs=pl.BlockSpec((m_blk, D_OUT), lambda i: (i, 0)),
        out_shape=jax.ShapeDtypeStruct((N_EDGES, D_OUT), jnp.float32),
    )(edge_feat, s, we, b2d)


def kernel(x, edge_feat, edge_index, W, b):
    we = W[:D_EDGE]
    ws = W[D_EDGE:D_EDGE + D_FEAT]
    wt = W[D_EDGE + D_FEAT:]
    xs, xt = _project(x, ws, wt)
    s = _gather_sum(xs, xt, edge_index[0], edge_index[1])
    return _final(edge_feat, s, we, b.reshape(1, D_OUT))


# speedup vs baseline: 2.3065x; 1.0718x over previous
import functools

import jax
import jax.numpy as jnp
from jax import lax
from jax.experimental import pallas as pl
from jax.experimental.pallas import tpu as pltpu
from jax.experimental.pallas import tpu_sc as plsc

N_NODES = 10000
N_EDGES = 160000
D_FEAT = 256
D_EDGE = 16
D_OUT = 256

NC, NS, LANES = 2, 16, 16
NW = NC * NS
EPW = N_EDGES // NW
CHUNK = 40
N_CHUNKS = EPW // CHUNK
NBUF = 5
PREF = 3


def _gather_sum_body(xs_hbm, xt_hbm, icat_hbm, out_hbm,
                     idx, bufa, bufb, sem_i, sem_ga, sem_gb, sem_w):
    wid = lax.axis_index("s") * NC + lax.axis_index("c")

    def idx_copy(chunk, b):
        return pltpu.make_async_copy(icat_hbm.at[wid, chunk], idx.at[b],
                                     sem_i.at[b])

    def gather_a(chunk, b):
        return pltpu.make_async_copy(xs_hbm.at[idx.at[b, 0]], bufa.at[b],
                                     sem_ga.at[b])

    def gather_b(chunk, b):
        return pltpu.make_async_copy(xt_hbm.at[idx.at[b, 1]], bufb.at[b],
                                     sem_gb.at[b])

    def out_copy(chunk, b):
        return pltpu.make_async_copy(
            bufa.at[b], out_hbm.at[pl.ds(wid * EPW + chunk * CHUNK, CHUNK)],
            sem_w.at[b])

    for c in range(PREF + 1):
        idx_copy(c, c).start()
    for c in range(PREF):
        idx_copy(c, c).wait()
        gather_a(c, c).start()
        gather_b(c, c).start()

    @pl.loop(0, N_CHUNKS, step=NBUF)
    def outer(base_chunk):
        for b in range(NBUF):
            chunk = base_chunk + b
            gather_a(chunk, b).wait()
            gather_b(chunk, b).wait()

            def row_body(e, carry):
                for j in range(D_FEAT // LANES):
                    sl = pl.ds(j * LANES, LANES)
                    bufa[b, e, sl] = bufa[b, e, sl] + bufb[b, e, sl]
                return carry

            lax.fori_loop(0, CHUNK, row_body, 0)
            out_copy(chunk, b).start()

            nb = (b + PREF) % NBUF
            ib = (b + PREF + 1) % NBUF

            @pl.when(chunk + PREF + 1 < N_CHUNKS)
            def _prefetch_idx():
                idx_copy(chunk + PREF + 1, ib).start()

            @pl.when((chunk + PREF < N_CHUNKS) & (chunk >= NBUF - PREF))
            def _wait_writeout():
                out_copy(chunk - (NBUF - PREF), nb).wait()

            @pl.when(chunk + PREF < N_CHUNKS)
            def _prefetch_gathers():
                idx_copy(chunk + PREF, nb).wait()
                gather_a(chunk + PREF, nb).start()
                gather_b(chunk + PREF, nb).start()

    for b in range(NBUF):
        out_copy(N_CHUNKS - NBUF + b, b).wait()


_gather_sum = pl.kernel(
    _gather_sum_body,
    out_type=jax.ShapeDtypeStruct((N_EDGES, D_FEAT), jnp.float32),
    mesh=plsc.VectorSubcoreMesh(core_axis_name="c", subcore_axis_name="s"),
    scratch_types=[
        pltpu.VMEM((NBUF, 2, CHUNK), jnp.int32),
        pltpu.VMEM((NBUF, CHUNK, D_FEAT), jnp.float32),
        pltpu.VMEM((NBUF, CHUNK, D_FEAT), jnp.float32),
        pltpu.SemaphoreType.DMA((NBUF,)),
        pltpu.SemaphoreType.DMA((NBUF,)),
        pltpu.SemaphoreType.DMA((NBUF,)),
        pltpu.SemaphoreType.DMA((NBUF,)),
    ],
)


def _project_body(x_ref, ws_ref, wt_ref, xs_ref, xt_ref):
    xb = x_ref[...]
    xs_ref[...] = jnp.dot(xb, ws_ref[...], preferred_element_type=jnp.float32)
    xt_ref[...] = jnp.dot(xb, wt_ref[...], preferred_element_type=jnp.float32)


def _project(x, ws, wt):
    m_blk = 1000
    grid = (N_NODES // m_blk,)
    return pl.pallas_call(
        _project_body,
        grid=grid,
        in_specs=[
            pl.BlockSpec((m_blk, D_FEAT), lambda i: (i, 0)),
            pl.BlockSpec((D_FEAT, D_FEAT), lambda i: (0, 0)),
            pl.BlockSpec((D_FEAT, D_FEAT), lambda i: (0, 0)),
        ],
        out_specs=[
            pl.BlockSpec((m_blk, D_FEAT), lambda i: (i, 0)),
            pl.BlockSpec((m_blk, D_FEAT), lambda i: (i, 0)),
        ],
        out_shape=[
            jax.ShapeDtypeStruct((N_NODES, D_FEAT), jnp.float32),
            jax.ShapeDtypeStruct((N_NODES, D_FEAT), jnp.float32),
        ],
    )(x, ws, wt)


def _final_body(ef_ref, s_ref, we_ref, b_ref, o_ref):
    acc = jnp.dot(ef_ref[...], we_ref[...], preferred_element_type=jnp.float32)
    o_ref[...] = jnp.maximum(acc + s_ref[...] + b_ref[...], 0.0)


def _final(edge_feat, s, we, b2d):
    m_blk = 2000
    grid = (N_EDGES // m_blk,)
    return pl.pallas_call(
        _final_body,
        grid=grid,
        in_specs=[
            pl.BlockSpec((m_blk, D_EDGE), lambda i: (i, 0)),
            pl.BlockSpec((m_blk, D_OUT), lambda i: (i, 0)),
            pl.BlockSpec((D_EDGE, D_OUT), lambda i: (0, 0)),
            pl.BlockSpec((1, D_OUT), lambda i: (0, 0)),
        ],
        out_specs=pl.BlockSpec((m_blk, D_OUT), lambda i: (i, 0)),
        out_shape=jax.ShapeDtypeStruct((N_EDGES, D_OUT), jnp.float32),
    )(edge_feat, s, we, b2d)


def kernel(x, edge_feat, edge_index, W, b):
    we = W[:D_EDGE]
    ws = W[D_EDGE:D_EDGE + D_FEAT]
    wt = W[D_EDGE + D_FEAT:]
    xs, xt = _project(x, ws, wt)
    icat = edge_index.reshape(2, NW, N_CHUNKS, CHUNK).transpose(1, 2, 0, 3)
    s = _gather_sum(xs, xt, icat)
    return _final(edge_feat, s, we, b.reshape(1, D_OUT))
